# Initial kernel scaffold; baseline (speedup 1.0000x reference)
import jax
import jax.numpy as jnp
from jax.experimental import pallas as pl


def kernel(confidence, predicted_locations, labels, gt_locations):
    raise NotImplementedError("write your pallas kernel here")


# speedup vs baseline: 1.1155x; 1.1155x over previous
import jax
import jax.numpy as jnp
from jax.experimental import pallas as pl

_NEG_POS_RATIO = 3


def _stage1(conf_ref, pred_ref, gt_ref, lab_ref, negv_ref, posce_ref, sl1_ref):
    c = conf_ref[0]
    lab = lab_ref[0, 0]
    m = jnp.max(c, axis=1)
    e = jnp.exp(c - m[:, None])
    lse = m + jnp.log(jnp.sum(e, axis=1))
    iota_c = jax.lax.broadcasted_iota(jnp.int32, c.shape, 1)
    c_at_label = jnp.sum(jnp.where(iota_c == lab[:, None], c, 0.0), axis=1)
    pos = lab > 0
    negv_ref[0, 0] = lse - c[:, 0]
    posce_ref[0, 0] = jnp.where(pos, lse - c_at_label, 0.0)
    d = pred_ref[0] - gt_ref[0]
    ad = jnp.abs(d)
    sl1 = jnp.where(ad < 1.0, 0.5 * d * d, ad - 0.5)
    sl1_ref[0, 0] = jnp.where(pos, jnp.sum(sl1, axis=1), 0.0)


def _stage2(negv_ref, posce_ref, sl1_ref, lab_ref, out_ref):
    nv = negv_ref[:, 0, :]
    lab = lab_ref[:, 0, :]
    pos = lab > 0
    npos = jnp.sum(pos.astype(jnp.int32), axis=1, keepdims=True)
    nneg = nv.shape[1] - npos
    k = jnp.minimum(_NEG_POS_RATIO * npos, nneg)
    u = jnp.where(pos, 0, jax.lax.bitcast_convert_type(nv, jnp.int32))

    def body(i, prefix):
        test = prefix | jnp.left_shift(jnp.int32(1), 30 - i)
        cnt = jnp.sum((u >= test).astype(jnp.int32), axis=1, keepdims=True)
        return jnp.where(cnt >= k, test, prefix)

    t = jax.lax.fori_loop(0, 31, body, jnp.zeros_like(k))
    gt_mask = u > t
    cnt_gt = jnp.sum(gt_mask.astype(jnp.int32), axis=1, keepdims=True)
    sum_gt = jnp.sum(jnp.where(gt_mask, nv, 0.0), axis=1, keepdims=True)
    tf = jnp.where(k > 0, jax.lax.bitcast_convert_type(t, jnp.float32), 0.0)
    topk = jnp.where(k > 0, sum_gt + (k - cnt_gt).astype(jnp.float32) * tf, 0.0)

    class_total = jnp.sum(posce_ref[:, 0, :]) + jnp.sum(topk)
    sl1_total = jnp.sum(sl1_ref[:, 0, :])
    nptot = jnp.sum(npos).astype(jnp.float32)
    lb = sl1_total / nptot
    lc = class_total / nptot
    lanes = jax.lax.broadcasted_iota(jnp.int32, (1, 128), 1)
    out_ref[...] = jnp.where(
        lanes == 0, lb + lc, jnp.where(lanes == 1, lb, jnp.where(lanes == 2, lc, 0.0))
    )


def kernel(confidence, predicted_locations, labels, gt_locations):
    B, P, C = confidence.shape
    lab3 = labels.astype(jnp.int32).reshape(B, 1, P)

    negv, posce, sl1 = pl.pallas_call(
        _stage1,
        grid=(B,),
        in_specs=[
            pl.BlockSpec((1, P, C), lambda b: (b, 0, 0)),
            pl.BlockSpec((1, P, 4), lambda b: (b, 0, 0)),
            pl.BlockSpec((1, P, 4), lambda b: (b, 0, 0)),
            pl.BlockSpec((1, 1, P), lambda b: (b, 0, 0)),
        ],
        out_specs=[
            pl.BlockSpec((1, 1, P), lambda b: (b, 0, 0)),
            pl.BlockSpec((1, 1, P), lambda b: (b, 0, 0)),
            pl.BlockSpec((1, 1, P), lambda b: (b, 0, 0)),
        ],
        out_shape=[
            jax.ShapeDtypeStruct((B, 1, P), jnp.float32),
            jax.ShapeDtypeStruct((B, 1, P), jnp.float32),
            jax.ShapeDtypeStruct((B, 1, P), jnp.float32),
        ],
    )(confidence, predicted_locations, gt_locations, lab3)

    out = pl.pallas_call(
        _stage2,
        out_shape=jax.ShapeDtypeStruct((1, 128), jnp.float32),
    )(negv, posce, sl1, lab3)

    return (out[0, 0], out[0, 1], out[0, 2])


# speedup vs baseline: 4.7556x; 4.2631x over previous
import jax
import jax.numpy as jnp
from jax.experimental import pallas as pl

_NEG_POS_RATIO = 3


def _stage1(conf_ref, pred_ref, gt_ref, lab_ref, negv_ref, posce_ref, sl1_ref):
    c = conf_ref[0]
    lab = lab_ref[0, 0]
    m = jnp.max(c, axis=0)
    e = jnp.exp(c - m[None, :])
    lse = m + jnp.log(jnp.sum(e, axis=0))
    iota_c = jax.lax.broadcasted_iota(jnp.int32, c.shape, 0)
    c_at_label = jnp.sum(jnp.where(iota_c == lab[None, :], c, 0.0), axis=0)
    pos = lab > 0
    negv_ref[0, 0] = lse - c[0, :]
    posce_ref[0, 0] = jnp.where(pos, lse - c_at_label, 0.0)
    d = pred_ref[0] - gt_ref[0]
    ad = jnp.abs(d)
    sl1 = jnp.where(ad < 1.0, 0.5 * d * d, ad - 0.5)
    sl1_ref[0, 0] = jnp.where(pos, jnp.sum(sl1, axis=0), 0.0)


def _stage2(negv_ref, posce_ref, sl1_ref, lab_ref, out_ref):
    nv = negv_ref[:, 0, :]
    lab = lab_ref[:, 0, :]
    pos = lab > 0
    npos = jnp.sum(pos.astype(jnp.int32), axis=1, keepdims=True)
    nneg = nv.shape[1] - npos
    k = jnp.minimum(_NEG_POS_RATIO * npos, nneg)
    u = jnp.where(pos, 0, jax.lax.bitcast_convert_type(nv, jnp.int32))

    def body(i, prefix):
        test = prefix | jnp.left_shift(jnp.int32(1), 30 - i)
        cnt = jnp.sum((u >= test).astype(jnp.int32), axis=1, keepdims=True)
        return jnp.where(cnt >= k, test, prefix)

    t = jax.lax.fori_loop(0, 31, body, jnp.zeros_like(k))
    gt_mask = u > t
    cnt_gt = jnp.sum(gt_mask.astype(jnp.int32), axis=1, keepdims=True)
    sum_gt = jnp.sum(jnp.where(gt_mask, nv, 0.0), axis=1, keepdims=True)
    tf = jnp.where(k > 0, jax.lax.bitcast_convert_type(t, jnp.float32), 0.0)
    topk = jnp.where(k > 0, sum_gt + (k - cnt_gt).astype(jnp.float32) * tf, 0.0)

    class_total = jnp.sum(posce_ref[:, 0, :]) + jnp.sum(topk)
    sl1_total = jnp.sum(sl1_ref[:, 0, :])
    nptot = jnp.sum(npos).astype(jnp.float32)
    lb = sl1_total / nptot
    lc = class_total / nptot
    lanes = jax.lax.broadcasted_iota(jnp.int32, (1, 128), 1)
    out_ref[...] = jnp.where(
        lanes == 0, lb + lc, jnp.where(lanes == 1, lb, jnp.where(lanes == 2, lc, 0.0))
    )


def kernel(confidence, predicted_locations, labels, gt_locations):
    B, P, C = confidence.shape
    lab3 = labels.astype(jnp.int32).reshape(B, 1, P)
    conf_t = jnp.transpose(confidence, (0, 2, 1))
    pred_t = jnp.transpose(predicted_locations, (0, 2, 1))
    gt_t = jnp.transpose(gt_locations, (0, 2, 1))

    negv, posce, sl1 = pl.pallas_call(
        _stage1,
        grid=(B,),
        in_specs=[
            pl.BlockSpec((1, C, P), lambda b: (b, 0, 0)),
            pl.BlockSpec((1, 4, P), lambda b: (b, 0, 0)),
            pl.BlockSpec((1, 4, P), lambda b: (b, 0, 0)),
            pl.BlockSpec((1, 1, P), lambda b: (b, 0, 0)),
        ],
        out_specs=[
            pl.BlockSpec((1, 1, P), lambda b: (b, 0, 0)),
            pl.BlockSpec((1, 1, P), lambda b: (b, 0, 0)),
            pl.BlockSpec((1, 1, P), lambda b: (b, 0, 0)),
        ],
        out_shape=[
            jax.ShapeDtypeStruct((B, 1, P), jnp.float32),
            jax.ShapeDtypeStruct((B, 1, P), jnp.float32),
            jax.ShapeDtypeStruct((B, 1, P), jnp.float32),
        ],
    )(conf_t, pred_t, gt_t, lab3)

    out = pl.pallas_call(
        _stage2,
        out_shape=jax.ShapeDtypeStruct((1, 128), jnp.float32),
    )(negv, posce, sl1, lab3)

    return (out[0, 0], out[0, 1], out[0, 2])


# speedup vs baseline: 4.7836x; 1.0059x over previous
import jax
import jax.numpy as jnp
from jax.experimental import pallas as pl

_NEG_POS_RATIO = 3


def _stage1(conf_ref, pred_ref, gt_ref, lab_ref, negv_ref, posce_ref, sl1_ref):
    c = conf_ref[0]
    lab = lab_ref[0, 0]
    m = jnp.max(c, axis=0)
    e = jnp.exp(c - m[None, :])
    lse = m + jnp.log(jnp.sum(e, axis=0))
    iota_c = jax.lax.broadcasted_iota(jnp.int32, c.shape, 0)
    c_at_label = jnp.sum(jnp.where(iota_c == lab[None, :], c, 0.0), axis=0)
    pos = lab > 0
    negv_ref[0, 0] = lse - c[0, :]
    posce_ref[0, 0] = jnp.where(pos, lse - c_at_label, 0.0)
    d = pred_ref[0] - gt_ref[0]
    ad = jnp.abs(d)
    sl1 = jnp.where(ad < 1.0, 0.5 * d * d, ad - 0.5)
    sl1_ref[0, 0] = jnp.where(pos, jnp.sum(sl1, axis=0), 0.0)


def _stage2(negv_ref, posce_ref, sl1_ref, lab_ref, out_ref):
    nv = negv_ref[:, 0, :]
    lab = lab_ref[:, 0, :]
    pos = lab > 0
    npos = jnp.sum(pos.astype(jnp.int32), axis=1, keepdims=True)
    nneg = nv.shape[1] - npos
    k = jnp.minimum(_NEG_POS_RATIO * npos, nneg)
    u = jnp.where(pos, 0, jax.lax.bitcast_convert_type(nv, jnp.int32))

    def body(i, prefix):
        test = prefix | jnp.left_shift(jnp.int32(1), 30 - i)
        cnt = jnp.sum((u >= test).astype(jnp.int32), axis=1, keepdims=True)
        return jnp.where(cnt >= k, test, prefix)

    t = jax.lax.fori_loop(0, 31, body, jnp.zeros_like(k))
    gt_mask = u > t
    cnt_gt = jnp.sum(gt_mask.astype(jnp.int32), axis=1, keepdims=True)
    sum_gt = jnp.sum(jnp.where(gt_mask, nv, 0.0), axis=1, keepdims=True)
    tf = jnp.where(k > 0, jax.lax.bitcast_convert_type(t, jnp.float32), 0.0)
    topk = jnp.where(k > 0, sum_gt + (k - cnt_gt).astype(jnp.float32) * tf, 0.0)

    class_total = jnp.sum(posce_ref[:, 0, :]) + jnp.sum(topk)
    sl1_total = jnp.sum(sl1_ref[:, 0, :])
    nptot = jnp.sum(npos).astype(jnp.float32)
    lb = sl1_total / nptot
    lc = class_total / nptot
    lanes = jax.lax.broadcasted_iota(jnp.int32, (1, 128), 1)
    out_ref[...] = jnp.where(
        lanes == 0, lb + lc, jnp.where(lanes == 1, lb, jnp.where(lanes == 2, lc, 0.0))
    )


def kernel(confidence, predicted_locations, labels, gt_locations):
    B, P, C = confidence.shape
    lab3 = labels.astype(jnp.int32).reshape(B, 1, P)
    conf_t = jnp.transpose(confidence, (0, 2, 1)) + 0.0
    pred_t = jnp.transpose(predicted_locations, (0, 2, 1)) + 0.0
    gt_t = jnp.transpose(gt_locations, (0, 2, 1)) + 0.0

    negv, posce, sl1 = pl.pallas_call(
        _stage1,
        grid=(B,),
        in_specs=[
            pl.BlockSpec((1, C, P), lambda b: (b, 0, 0)),
            pl.BlockSpec((1, 4, P), lambda b: (b, 0, 0)),
            pl.BlockSpec((1, 4, P), lambda b: (b, 0, 0)),
            pl.BlockSpec((1, 1, P), lambda b: (b, 0, 0)),
        ],
        out_specs=[
            pl.BlockSpec((1, 1, P), lambda b: (b, 0, 0)),
            pl.BlockSpec((1, 1, P), lambda b: (b, 0, 0)),
            pl.BlockSpec((1, 1, P), lambda b: (b, 0, 0)),
        ],
        out_shape=[
            jax.ShapeDtypeStruct((B, 1, P), jnp.float32),
            jax.ShapeDtypeStruct((B, 1, P), jnp.float32),
            jax.ShapeDtypeStruct((B, 1, P), jnp.float32),
        ],
    )(conf_t, pred_t, gt_t, lab3)

    out = pl.pallas_call(
        _stage2,
        out_shape=jax.ShapeDtypeStruct((1, 128), jnp.float32),
    )(negv, posce, sl1, lab3)

    return (out[0, 0], out[0, 1], out[0, 2])


# speedup vs baseline: 5.1748x; 1.0818x over previous
import jax
import jax.numpy as jnp
from jax.experimental import pallas as pl

_NEG_POS_RATIO = 3


def _stage1(conf_ref, pred_ref, gt_ref, lab_ref, negv_ref, posce_ref, sl1_ref):
    c = conf_ref[0]
    lab = lab_ref[0, 0]
    m = jnp.max(c, axis=0)
    e = jnp.exp(c - m[None, :])
    lse = m + jnp.log(jnp.sum(e, axis=0))
    iota_c = jax.lax.broadcasted_iota(jnp.int32, c.shape, 0)
    c_at_label = jnp.sum(jnp.where(iota_c == lab[None, :], c, 0.0), axis=0)
    pos = lab > 0
    negv_ref[0, 0] = lse - c[0, :]
    posce_ref[0, 0] = jnp.where(pos, lse - c_at_label, 0.0)
    d = pred_ref[0] - gt_ref[0]
    ad = jnp.abs(d)
    sl1 = jnp.where(ad < 1.0, 0.5 * d * d, ad - 0.5)
    sl1_ref[0, 0] = jnp.where(pos, jnp.sum(sl1, axis=0), 0.0)


def _stage2(negv_ref, posce_ref, sl1_ref, lab_ref, out_ref):
    nv = negv_ref[:, 0, :]
    lab = lab_ref[:, 0, :]
    pos = lab > 0
    npos = jnp.sum(pos.astype(jnp.int32), axis=1, keepdims=True)
    nneg = nv.shape[1] - npos
    k = jnp.minimum(_NEG_POS_RATIO * npos, nneg)
    u = jnp.where(pos, 0, jax.lax.bitcast_convert_type(nv, jnp.int32))

    def body(i, prefix):
        test = prefix | jnp.left_shift(jnp.int32(1), 30 - i)
        cnt = jnp.sum((u >= test).astype(jnp.int32), axis=1, keepdims=True)
        return jnp.where(cnt >= k, test, prefix)

    t = jax.lax.fori_loop(0, 31, body, jnp.zeros_like(k))
    gt_mask = u > t
    cnt_gt = jnp.sum(gt_mask.astype(jnp.int32), axis=1, keepdims=True)
    sum_gt = jnp.sum(jnp.where(gt_mask, nv, 0.0), axis=1, keepdims=True)
    tf = jnp.where(k > 0, jax.lax.bitcast_convert_type(t, jnp.float32), 0.0)
    topk = jnp.where(k > 0, sum_gt + (k - cnt_gt).astype(jnp.float32) * tf, 0.0)

    class_total = jnp.sum(posce_ref[:, 0, :]) + jnp.sum(topk)
    sl1_total = jnp.sum(sl1_ref[:, 0, :])
    nptot = jnp.sum(npos).astype(jnp.float32)
    lb = sl1_total / nptot
    lc = class_total / nptot
    lanes = jax.lax.broadcasted_iota(jnp.int32, (1, 128), 1)
    out_ref[...] = jnp.where(
        lanes == 0, lb + lc, jnp.where(lanes == 1, lb, jnp.where(lanes == 2, lc, 0.0))
    )


def kernel(confidence, predicted_locations, labels, gt_locations):
    B, P, C = confidence.shape
    lab3 = labels.astype(jnp.int32).reshape(B, 1, P)
    zero = jax.lax.optimization_barrier(jnp.float32(0.0))
    conf_t = jnp.transpose(confidence, (0, 2, 1)) + zero
    pred_t = jnp.transpose(predicted_locations, (0, 2, 1)) + zero
    gt_t = jnp.transpose(gt_locations, (0, 2, 1)) + zero

    negv, posce, sl1 = pl.pallas_call(
        _stage1,
        grid=(B,),
        in_specs=[
            pl.BlockSpec((1, C, P), lambda b: (b, 0, 0)),
            pl.BlockSpec((1, 4, P), lambda b: (b, 0, 0)),
            pl.BlockSpec((1, 4, P), lambda b: (b, 0, 0)),
            pl.BlockSpec((1, 1, P), lambda b: (b, 0, 0)),
        ],
        out_specs=[
            pl.BlockSpec((1, 1, P), lambda b: (b, 0, 0)),
            pl.BlockSpec((1, 1, P), lambda b: (b, 0, 0)),
            pl.BlockSpec((1, 1, P), lambda b: (b, 0, 0)),
        ],
        out_shape=[
            jax.ShapeDtypeStruct((B, 1, P), jnp.float32),
            jax.ShapeDtypeStruct((B, 1, P), jnp.float32),
            jax.ShapeDtypeStruct((B, 1, P), jnp.float32),
        ],
    )(conf_t, pred_t, gt_t, lab3)

    out = pl.pallas_call(
        _stage2,
        out_shape=jax.ShapeDtypeStruct((1, 128), jnp.float32),
    )(negv, posce, sl1, lab3)

    return (out[0, 0], out[0, 1], out[0, 2])
